# baseline (speedup 1.0000x reference)
import functools

import jax
import jax.numpy as jnp
import numpy as np
from jax import lax
from jax.experimental import pallas as pl
from jax.experimental.pallas import tpu as pltpu
from jax.experimental.pallas import tpu_sc as plsc

NUM_FIELDS = 26
EMBED_DIM = 16
BATCH = 16384
FIELD_SIZE = 100000
_OFFSETS = jnp.asarray(
    np.arange(NUM_FIELDS, dtype=np.int32) * FIELD_SIZE, dtype=jnp.int32
)

NUM_CORES = 2
NUM_SUBCORES = 16
NUM_WORKERS = NUM_CORES * NUM_SUBCORES
B_PER_W = BATCH // NUM_WORKERS
CHUNK_ROWS = 64
NUM_CHUNKS = B_PER_W // CHUNK_ROWS
CHUNK_IDX = CHUNK_ROWS * NUM_FIELDS


def _fm_body(idx_hbm, table_hbm, linw_hbm, out_hbm,
             idx_v, rows_v, lin_v, rmat, out_v, sem_e, sem_l):
    c = lax.axis_index("c")
    s = lax.axis_index("s")
    wid = s * NUM_CORES + c
    base = wid * B_PER_W
    lane = lax.iota(jnp.int32, 16)
    tail_mask = lane < (NUM_FIELDS - 16)

    def chunk_body(g, carry):
        gbase = base + g * CHUNK_ROWS
        pltpu.sync_copy(idx_hbm.at[pl.ds(gbase * NUM_FIELDS, CHUNK_IDX)], idx_v)
        cp_e = pltpu.async_copy(table_hbm.at[idx_v], rows_v, sem_e)
        cp_l = pltpu.async_copy(linw_hbm.at[idx_v], lin_v.at[pl.ds(0, CHUNK_IDX)],
                                sem_l)
        cp_e.wait()
        cp_l.wait()

        def group_body(gr, inner_carry):
            acc = jnp.zeros((16,), jnp.float32)
            for j in range(16):
                r = gr * 16 + j
                fbase = r * NUM_FIELDS
                v0 = rows_v[fbase, :]
                s_acc = v0
                q_acc = v0 * v0
                for f in range(1, NUM_FIELDS):
                    v = rows_v[fbase + f, :]
                    s_acc = s_acc + v
                    q_acc = q_acc + v * v
                l1 = lin_v[pl.ds(fbase, 16)]
                l2 = lin_v[pl.ds(fbase + 16, 16)]
                rmat[pl.ds(j * 16, 16)] = (
                    0.5 * (s_acc * s_acc - q_acc)
                    + l1 + jnp.where(tail_mask, l2, 0.0))
            col_base = lane * 16
            acc = plsc.load_gather(rmat, [col_base])
            for d in range(1, 16):
                acc = acc + plsc.load_gather(rmat, [col_base + d])
            out_v[pl.ds(g * CHUNK_ROWS + gr * 16, 16)] = acc
            return inner_carry

        lax.fori_loop(0, CHUNK_ROWS // 16, group_body, 0)
        return carry

    lax.fori_loop(0, NUM_CHUNKS, chunk_body, 0)
    pltpu.sync_copy(out_v, out_hbm.at[pl.ds(base, B_PER_W)])


@functools.partial(jax.jit, static_argnames=())
def _fm_call(idx_flat, embed_table, linw_flat):
    mesh = plsc.VectorSubcoreMesh(
        core_axis_name="c", subcore_axis_name="s",
        num_cores=NUM_CORES, num_subcores=NUM_SUBCORES,
    )
    run = pl.kernel(
        _fm_body,
        out_type=jax.ShapeDtypeStruct((BATCH,), jnp.float32),
        mesh=mesh,
        compiler_params=pltpu.CompilerParams(
            needs_layout_passes=False, use_tc_tiling_on_sc=False),
        scratch_types=[
            pltpu.VMEM((CHUNK_IDX,), jnp.int32),
            pltpu.VMEM((CHUNK_IDX, EMBED_DIM), jnp.float32),
            pltpu.VMEM((CHUNK_IDX + 16,), jnp.float32),
            pltpu.VMEM((256,), jnp.float32),
            pltpu.VMEM((B_PER_W,), jnp.float32),
            pltpu.SemaphoreType.DMA,
            pltpu.SemaphoreType.DMA,
        ],
    )
    return run(idx_flat, embed_table, linw_flat)


def kernel(x, embed_table, linear_w, bias):
    idx_flat = (x + _OFFSETS[None, :]).reshape(-1)
    out = _fm_call(idx_flat, embed_table, linear_w.reshape(-1))
    return out.reshape(BATCH, 1) + bias
